# baseline (speedup 1.0000x reference)
import jax
import jax.numpy as jnp
from jax import lax
from jax.experimental import pallas as pl
from jax.experimental.pallas import tpu as pltpu
from jax.experimental.pallas import tpu_sc as plsc

_B, _S = 16384, 100
_N = _B * _S
_V, _D = 100, 256
_NC, _NS = 2, 16
_NW = _NC * _NS
_ROWS_PER_W = _N // _NW
_CHUNK = 128
_CHUNKS = _ROWS_PER_W // _CHUNK
_LANES = 16

_MAGIC = 8388608.0


def _make_sc_gather():
    mesh = plsc.VectorSubcoreMesh(core_axis_name="c", subcore_axis_name="s")

    @pl.kernel(
        out_type=jax.ShapeDtypeStruct((_N, _D), jnp.float32),
        mesh=mesh,
        scratch_types=[
            pltpu.VMEM((_V + 4, _D), jnp.float32),
            pltpu.VMEM((_CHUNK,), jnp.float32),
            pltpu.VMEM((_CHUNK,), jnp.int32),
            pltpu.VMEM((_CHUNK, _D), jnp.float32),
            pltpu.VMEM((_CHUNK, _D), jnp.float32),
            pltpu.VMEM((_CHUNK, _D), jnp.float32),
            pltpu.SemaphoreType.DMA,
            pltpu.SemaphoreType.DMA,
            pltpu.SemaphoreType.DMA,
            pltpu.SemaphoreType.DMA,
        ],
    )
    def sc_gather(x_hbm, rep_hbm, out_hbm, tbl_v, x_v, idx_v,
                  rows_a, rows_b, rows_c, sga, swa, swb, swc):
        cid = lax.axis_index("c")
        sid = lax.axis_index("s")
        wid = cid * _NS + sid
        woff = wid * _ROWS_PER_W

        pltpu.sync_copy(rep_hbm.at[pl.ds(0, _V + 4)], tbl_v)

        def load_idx(c, off):
            pltpu.sync_copy(x_hbm.at[pl.ds(woff + c * _CHUNK, _CHUNK)], x_v)
            for i in range(_CHUNK // _LANES):
                sl = pl.ds(i * _LANES, _LANES)
                y = jnp.maximum(x_v[sl], 0.01) * 100.0
                r = (y + _MAGIC) - _MAGIC
                idx_v[sl] = r.astype(jnp.int32) + off

        def start_gather(c, rows):
            load_idx(c, wid * _V - 1)
            pltpu.async_copy(rep_hbm.at[idx_v], rows, sga)

        def wait_gather(rows):
            pltpu.make_async_copy(rep_hbm.at[idx_v], rows, sga).wait()

        def build_chunk(c, rows):
            load_idx(c, -1)
            @pl.loop(0, _CHUNK // _LANES)
            def _(g):
                iv = idx_v[pl.ds(g * _LANES, _LANES)]
                base = g * _LANES
                for lane in range(_LANES):
                    src = tbl_v.at[iv[lane]]
                    dst = rows.at[base + lane]
                    vals = [src[pl.ds(j * _LANES, _LANES)]
                            for j in range(_D // _LANES)]
                    for j in range(_D // _LANES):
                        dst[pl.ds(j * _LANES, _LANES)] = vals[j]

        def out_slice(c):
            return out_hbm.at[pl.ds(woff + c * _CHUNK, _CHUNK)]

        def start_write(c, rows, sw):
            pltpu.async_copy(rows, out_slice(c), sw)

        def wait_write(c, rows, sw):
            pltpu.make_async_copy(rows, out_slice(c), sw).wait()

        start_gather(0, rows_a)
        build_chunk(1, rows_b)
        start_write(1, rows_b, swb)
        build_chunk(2, rows_c)
        start_write(2, rows_c, swc)
        wait_gather(rows_a)
        start_write(0, rows_a, swa)

        @pl.loop(3, _CHUNKS - 1, step=3)
        def _(c):
            wait_write(c - 3, rows_a, swa)
            start_gather(c, rows_a)
            wait_write(c - 2, rows_b, swb)
            build_chunk(c + 1, rows_b)
            start_write(c + 1, rows_b, swb)
            wait_write(c - 1, rows_c, swc)
            build_chunk(c + 2, rows_c)
            start_write(c + 2, rows_c, swc)
            wait_gather(rows_a)
            start_write(c, rows_a, swa)

        wait_write(_CHUNKS - 4, rows_a, swa)
        build_chunk(_CHUNKS - 1, rows_a)
        start_write(_CHUNKS - 1, rows_a, swa)
        wait_write(_CHUNKS - 3, rows_b, swb)
        wait_write(_CHUNKS - 2, rows_c, swc)
        wait_write(_CHUNKS - 1, rows_a, swa)

    return sc_gather


_sc_gather = _make_sc_gather()


def kernel(x, pe):
    pe_rep = jnp.tile(pe, (_NW, 1))
    xt = x.T.reshape(_N)
    out = _sc_gather(xt, pe_rep)
    return out.reshape(_S, _B, _D).transpose(1, 0, 2)
